# baseline (speedup 1.0000x reference)
import jax
import jax.numpy as jnp
from jax.experimental import pallas as pl

L = 128
P = 4
U = 32
D = L * P
TB = 1024


def _fused(x_ref, ui_ref, mui_ref, w1_ref, b1_ref, w2_ref, b2_ref,
           wd1_ref, bd1_ref, wd2t_ref, bd2_ref, q_ref):
    x = x_ref[...]
    bf = lambda a: a.astype(jnp.bfloat16)
    h = jnp.tanh(jnp.dot(bf(x), bf(w1_ref[...]),
                         preferred_element_type=jnp.float32) + b1_ref[...])
    logits = jnp.dot(bf(h), bf(w2_ref[...]),
                     preferred_element_type=jnp.float32) + b2_ref[...]
    m = jnp.max(logits, axis=1, keepdims=True)
    e = jnp.exp(logits - m)
    y = e / jnp.sum(e, axis=1, keepdims=True)

    turn = x[:, P:2 * P]
    budgets = jnp.max(x[:, 0:P], axis=1, keepdims=True)
    pid = jnp.argmax(turn, axis=1)[:, None]

    lcol = jax.lax.broadcasted_iota(jnp.int32, (U, D), 1) // P
    g = ((ui_ref[...] == lcol).astype(jnp.float32)
         + (mui_ref[...] == lcol).astype(jnp.float32))
    s_exp = jnp.dot(bf(y), bf(g), preferred_element_type=jnp.float32)
    pcol = jax.lax.broadcasted_iota(jnp.int32, (P, D), 1) % P
    prow = jax.lax.broadcasted_iota(jnp.int32, (P, D), 0)
    f = (pcol == prow).astype(jnp.float32)
    tt_exp = jnp.dot(bf(budgets * turn), bf(f),
                     preferred_element_type=jnp.float32)
    x2 = x + s_exp * tt_exp

    p4 = jax.lax.broadcasted_iota(jnp.int32, (TB, P), 1)
    masked = jnp.where(p4 == pid, x[:, 0:P], 0.0)
    corr = jnp.dot(masked, wd1_ref[0:P, :],
                   preferred_element_type=jnp.float32)

    hd = jnp.tanh(jnp.dot(bf(x2), bf(wd1_ref[...]),
                          preferred_element_type=jnp.float32)
                  - corr + bd1_ref[...])
    w = jnp.dot(turn, wd2t_ref[...],
                preferred_element_type=jnp.float32)
    qb = jnp.dot(turn, bd2_ref[...],
                 preferred_element_type=jnp.float32)
    q_ref[...] = jnp.sum(hd * w, axis=1, keepdims=True) + qb


def kernel(data, unit_indices, moveable_unit_indices,
           W1, b1, W2, b2, Wd1, bd1, Wd2, bd2):
    batch = data.shape[0]
    ui = unit_indices.astype(jnp.int32).reshape(U, 1)
    mui = moveable_unit_indices.astype(jnp.int32).reshape(U, 1)
    rep = lambda shape: pl.BlockSpec(shape, lambda i: (0, 0))
    q = pl.pallas_call(
        _fused,
        grid=(batch // TB,),
        in_specs=[
            pl.BlockSpec((TB, D), lambda i: (i, 0)),
            rep((U, 1)), rep((U, 1)),
            rep((D, H := W1.shape[1])), rep((1, H)),
            rep((H, U)), rep((1, U)),
            rep((D, H)), rep((1, H)),
            rep((P, H)), rep((P, 1)),
        ],
        out_specs=pl.BlockSpec((TB, 1), lambda i: (i, 0)),
        out_shape=jax.ShapeDtypeStruct((batch, 1), jnp.float32),
    )(data, ui, mui, W1, b1.reshape(1, -1), W2, b2.reshape(1, -1),
      Wd1, bd1.reshape(1, -1), Wd2.T, bd2.reshape(P, 1))
    return q.reshape(batch)


# speedup vs baseline: 1.1241x; 1.1241x over previous
import jax
import jax.numpy as jnp
from jax.experimental import pallas as pl

L = 128
P = 4
U = 32
D = L * P
TB = 1024


def _fused(x_ref, ui_ref, mui_ref, w1_ref, b1_ref, w2_ref, b2_ref,
           wd1_ref, bd1_ref, wd2_ref, bd2_ref, q_ref):
    x = x_ref[...]
    bf = lambda a: a.astype(jnp.bfloat16)
    h = jnp.tanh(jnp.dot(bf(x), bf(w1_ref[...]),
                         preferred_element_type=jnp.float32) + b1_ref[...])
    logits = jnp.dot(bf(h), bf(w2_ref[...]),
                     preferred_element_type=jnp.float32) + b2_ref[...]
    m = jnp.max(logits, axis=1, keepdims=True)
    e = jnp.exp(logits - m)
    y = e / jnp.sum(e, axis=1, keepdims=True)

    turn = x[:, P:2 * P]
    budgets = jnp.max(x[:, 0:P], axis=1, keepdims=True)
    pid = jnp.argmax(turn, axis=1)[:, None]

    j = jax.lax.broadcasted_iota(jnp.int32, (TB, D), 1)
    z = jnp.where(j == pid, 0.0, x)

    lcol = jax.lax.broadcasted_iota(jnp.int32, (U, D), 1) // P
    g = ((ui_ref[...] == lcol).astype(jnp.float32)
         + (mui_ref[...] == lcol).astype(jnp.float32))
    s_exp = jnp.dot(bf(y), bf(g), preferred_element_type=jnp.float32)
    pcol = jax.lax.broadcasted_iota(jnp.int32, (P, D), 1) % P
    prow = jax.lax.broadcasted_iota(jnp.int32, (P, D), 0)
    f = (pcol == prow).astype(jnp.float32)
    tt_exp = jnp.dot(bf(budgets * turn), bf(f),
                     preferred_element_type=jnp.float32)
    x2 = z + s_exp * tt_exp

    hd = jnp.tanh(jnp.dot(bf(x2), bf(wd1_ref[...]),
                          preferred_element_type=jnp.float32) + bd1_ref[...])
    out = jnp.dot(bf(hd), bf(wd2_ref[...]),
                  preferred_element_type=jnp.float32) + bd2_ref[...]
    q_ref[...] = jnp.sum(out * turn, axis=1, keepdims=True)


def kernel(data, unit_indices, moveable_unit_indices,
           W1, b1, W2, b2, Wd1, bd1, Wd2, bd2):
    batch = data.shape[0]
    ui = unit_indices.astype(jnp.int32).reshape(U, 1)
    mui = moveable_unit_indices.astype(jnp.int32).reshape(U, 1)
    rep = lambda shape: pl.BlockSpec(shape, lambda i: (0, 0))
    q = pl.pallas_call(
        _fused,
        grid=(batch // TB,),
        in_specs=[
            pl.BlockSpec((TB, D), lambda i: (i, 0)),
            rep((U, 1)), rep((U, 1)),
            rep((D, H := W1.shape[1])), rep((1, H)),
            rep((H, U)), rep((1, U)),
            rep((D, H)), rep((1, H)),
            rep((H, P)), rep((1, P)),
        ],
        out_specs=pl.BlockSpec((TB, 1), lambda i: (i, 0)),
        out_shape=jax.ShapeDtypeStruct((batch, 1), jnp.float32),
    )(data, ui, mui, W1, b1.reshape(1, -1), W2, b2.reshape(1, -1),
      Wd1, bd1.reshape(1, -1), Wd2, bd2.reshape(1, -1))
    return q.reshape(batch)


# speedup vs baseline: 1.2025x; 1.0697x over previous
import jax
import jax.numpy as jnp
from jax.experimental import pallas as pl

L = 128
P = 4
U = 32
D = L * P
TB = 2048


def _fused(x_ref, ui_ref, mui_ref, w1_ref, b1_ref, w2_ref, b2_ref,
           wd1_ref, bd1_ref, wd2_ref, bd2_ref, q_ref):
    x = x_ref[...]
    bf = lambda a: a.astype(jnp.bfloat16)
    h = jnp.tanh(jnp.dot(bf(x), bf(w1_ref[...]),
                         preferred_element_type=jnp.float32) + b1_ref[...])
    logits = jnp.dot(bf(h), bf(w2_ref[...]),
                     preferred_element_type=jnp.float32) + b2_ref[...]
    m = jnp.max(logits, axis=1, keepdims=True)
    e = jnp.exp(logits - m)
    y = e / jnp.sum(e, axis=1, keepdims=True)

    turn = x[:, P:2 * P]
    budgets = jnp.max(x[:, 0:P], axis=1, keepdims=True)
    pid = jnp.argmax(turn, axis=1)[:, None]

    j = jax.lax.broadcasted_iota(jnp.int32, (TB, D), 1)
    z = jnp.where(j == pid, 0.0, x)

    lcol = jax.lax.broadcasted_iota(jnp.int32, (U, D), 1) // P
    g = ((ui_ref[...] == lcol).astype(jnp.float32)
         + (mui_ref[...] == lcol).astype(jnp.float32))
    s_exp = jnp.dot(bf(y), bf(g), preferred_element_type=jnp.float32)
    pcol = jax.lax.broadcasted_iota(jnp.int32, (P, D), 1) % P
    prow = jax.lax.broadcasted_iota(jnp.int32, (P, D), 0)
    f = (pcol == prow).astype(jnp.float32)
    tt_exp = jnp.dot(bf(budgets * turn), bf(f),
                     preferred_element_type=jnp.float32)
    x2 = z + s_exp * tt_exp

    hd = jnp.tanh(jnp.dot(bf(x2), bf(wd1_ref[...]),
                          preferred_element_type=jnp.float32) + bd1_ref[...])
    out = jnp.dot(bf(hd), bf(wd2_ref[...]),
                  preferred_element_type=jnp.float32) + bd2_ref[...]
    q_ref[...] = jnp.sum(out * turn, axis=1, keepdims=True)


def kernel(data, unit_indices, moveable_unit_indices,
           W1, b1, W2, b2, Wd1, bd1, Wd2, bd2):
    batch = data.shape[0]
    ui = unit_indices.astype(jnp.int32).reshape(U, 1)
    mui = moveable_unit_indices.astype(jnp.int32).reshape(U, 1)
    rep = lambda shape: pl.BlockSpec(shape, lambda i: (0, 0))
    q = pl.pallas_call(
        _fused,
        grid=(batch // TB,),
        in_specs=[
            pl.BlockSpec((TB, D), lambda i: (i, 0)),
            rep((U, 1)), rep((U, 1)),
            rep((D, H := W1.shape[1])), rep((1, H)),
            rep((H, U)), rep((1, U)),
            rep((D, H)), rep((1, H)),
            rep((H, P)), rep((1, P)),
        ],
        out_specs=pl.BlockSpec((TB, 1), lambda i: (i, 0)),
        out_shape=jax.ShapeDtypeStruct((batch, 1), jnp.float32),
    )(data, ui, mui, W1, b1.reshape(1, -1), W2, b2.reshape(1, -1),
      Wd1, bd1.reshape(1, -1), Wd2, bd2.reshape(1, -1))
    return q.reshape(batch)
